# Initial kernel scaffold; baseline (speedup 1.0000x reference)
import jax
import jax.numpy as jnp
from jax.experimental import pallas as pl


def kernel(logits, k):
    raise NotImplementedError("write your pallas kernel here")


# speedup vs baseline: 2.0509x; 2.0509x over previous
import functools

import jax
import jax.numpy as jnp
from jax import lax
from jax.experimental import pallas as pl
from jax.experimental.pallas import tpu as pltpu

_K = 64
_ROWS_PER_STEP = 8


def _threefry_bits(p):
    ks = (jnp.uint32(0), jnp.uint32(1), jnp.uint32(0x1BD11BDA ^ 1))
    rotations = ((13, 15, 26, 6), (17, 29, 16, 24))
    x0 = jnp.zeros_like(p)
    x1 = p + ks[1]
    for i in range(5):
        for r in rotations[i % 2]:
            x0 = x0 + x1
            x1 = (x1 << jnp.uint32(r)) | (x1 >> jnp.uint32(32 - r))
            x1 = x1 ^ x0
        x0 = x0 + ks[(i + 1) % 3]
        x1 = x1 + ks[(i + 2) % 3] + jnp.uint32(i + 1)
    return x0 ^ x1


def _step(x_ref, probs_ref, tok_ref, *, vocab, rows):
    x = x_ref[...]
    xu = lax.bitcast_convert_type(x, jnp.uint32)
    y = jnp.where((xu >> jnp.uint32(31)) == 0, xu | jnp.uint32(0x80000000), ~xu)

    def bs_body(i, u):
        bit = lax.shift_left(jnp.uint32(1), (31 - i).astype(jnp.uint32))
        cand = u | bit
        cnt = jnp.sum((y >= cand).astype(jnp.int32), axis=1, keepdims=True)
        return jnp.where(cnt >= _K, cand, u)

    u = lax.fori_loop(0, 32, bs_body, jnp.zeros((rows, 1), jnp.uint32))

    gt = y > u
    n_gt = jnp.sum(gt.astype(jnp.int32), axis=1, keepdims=True)
    r = _K - n_gt
    eq = y == u
    n_eq = jnp.sum(eq.astype(jnp.int32), axis=1, keepdims=True)
    col = lax.broadcasted_iota(jnp.int32, x.shape, 1)

    def col_search(_):
        def cs_body(i, t):
            bit = lax.shift_left(jnp.int32(1), (16 - i))
            cand = t | bit
            cnt = jnp.sum((eq & (col <= cand)).astype(jnp.int32),
                          axis=1, keepdims=True)
            return jnp.where(cnt <= r, cand, t)

        return lax.fori_loop(0, 17, cs_body, jnp.zeros((rows, 1), jnp.int32))

    t = lax.cond(jnp.any(n_eq != r), col_search,
                 lambda _: jnp.full((rows, 1), jnp.int32(0x7FFFFFFF)), None)

    sel = gt | (eq & (col <= t))
    m = jnp.max(x, axis=1, keepdims=True)
    e = jnp.where(sel, jnp.exp(x - m), jnp.float32(0.0))
    z = jnp.sum(e, axis=1, keepdims=True)
    probs_ref[...] = e / z

    row0 = pl.program_id(0) * rows
    rowid = row0 + lax.broadcasted_iota(jnp.int32, (rows, 1), 0)
    p = (rowid * vocab + col).astype(jnp.uint32)
    bits = _threefry_bits(p)
    fb = (bits >> jnp.uint32(9)) | jnp.uint32(0x3F800000)
    f = lax.bitcast_convert_type(fb, jnp.float32) - jnp.float32(1.0)
    tiny = jnp.float32(1.1754943508222875e-38)
    u01 = jnp.maximum(tiny, f * (jnp.float32(1.0) - tiny) + tiny)
    g = -jnp.log(-jnp.log(u01))

    score = jnp.where(sel, x + g, -jnp.inf)
    smax = jnp.max(score, axis=1, keepdims=True)
    tok = jnp.min(jnp.where(score == smax, col, jnp.int32(0x7FFFFFFF)),
                  axis=1, keepdims=True)
    tok_ref[...] = tok


def kernel(logits, k):
    del k
    b, vocab = logits.shape
    rows = _ROWS_PER_STEP
    grid = b // rows
    probs, tok = pl.pallas_call(
        functools.partial(_step, vocab=vocab, rows=rows),
        grid=(grid,),
        in_specs=[pl.BlockSpec((rows, vocab), lambda i: (i, 0))],
        out_specs=[
            pl.BlockSpec((rows, vocab), lambda i: (i, 0)),
            pl.BlockSpec((rows, 1), lambda i: (i, 0)),
        ],
        out_shape=[
            jax.ShapeDtypeStruct((b, vocab), jnp.float32),
            jax.ShapeDtypeStruct((b, 1), jnp.int32),
        ],
    )(logits)
    return tok.reshape(b), probs


# speedup vs baseline: 2.5086x; 1.2231x over previous
import functools

import jax
import jax.numpy as jnp
from jax import lax
from jax.experimental import pallas as pl
from jax.experimental.pallas import tpu as pltpu

_K = 64
_ROWS_PER_STEP = 8


def _threefry_bits(p):
    ks = (jnp.uint32(0), jnp.uint32(1), jnp.uint32(0x1BD11BDA ^ 1))
    rotations = ((13, 15, 26, 6), (17, 29, 16, 24))
    x0 = jnp.zeros_like(p)
    x1 = p + ks[1]
    for i in range(5):
        for r in rotations[i % 2]:
            x0 = x0 + x1
            x1 = (x1 << jnp.uint32(r)) | (x1 >> jnp.uint32(32 - r))
            x1 = x1 ^ x0
        x0 = x0 + ks[(i + 1) % 3]
        x1 = x1 + ks[(i + 2) % 3] + jnp.uint32(i + 1)
    return x0 ^ x1


def _count_ge(v16, cand16, rows, pcols):
    msk = (v16 >= cand16).astype(jnp.int16)
    accs = [jnp.zeros((rows, 256), jnp.int16) for _ in range(4)]
    for j in range(pcols // 256):
        accs[j % 4] = accs[j % 4] + msk[:, j * 256:(j + 1) * 256]
    part = (accs[0] + accs[1]) + (accs[2] + accs[3])
    return jnp.sum(part.astype(jnp.int32), axis=1, keepdims=True)


def _step(x_ref, probs_ref, tok_ref, *, vocab, rows):
    x = x_ref[...]
    xu = lax.bitcast_convert_type(x, jnp.uint32)
    y = jnp.where((xu >> jnp.uint32(31)) == 0, xu | jnp.uint32(0x80000000), ~xu)

    pcols = -(-vocab // 256) * 256
    pad = pcols - vocab
    bias = jnp.int32(0x8000)

    def pack16(v32):
        v = jnp.concatenate(
            [v32, jnp.zeros((rows, pad), jnp.int32)], axis=1)
        return (v ^ bias).astype(jnp.int16)

    hi = lax.bitcast_convert_type(y >> jnp.uint32(16), jnp.int32)
    y16 = pack16(hi)

    def p1_body(i, p):
        cand = p | lax.shift_left(jnp.int32(1), 15 - i)
        cnt = _count_ge(y16, (cand ^ bias).astype(jnp.int16),
                        rows, pcols)
        return jnp.where(cnt >= _K, cand, p)

    p16 = lax.fori_loop(0, 16, p1_body, jnp.zeros((rows, 1), jnp.int32))

    p16b = (p16 ^ bias).astype(jnp.int16)
    n_above = jnp.sum((y16 > p16b).astype(jnp.int32), axis=1, keepdims=True)
    eq16 = y16 == p16b
    lo = lax.bitcast_convert_type(y & jnp.uint32(0xFFFF), jnp.int32)
    key16 = pack16(jnp.where(eq16[:, :vocab], lo, jnp.int32(0)))

    def p2_body(i, c):
        cand = c | lax.shift_left(jnp.int32(1), 15 - i)
        cnt = n_above + _count_ge(
            key16, (cand ^ bias).astype(jnp.int16), rows, pcols)
        return jnp.where(cnt >= _K, cand, c)

    c16 = lax.fori_loop(0, 16, p2_body, jnp.zeros((rows, 1), jnp.int32))

    u = (lax.bitcast_convert_type(p16, jnp.uint32) << jnp.uint32(16)) | \
        lax.bitcast_convert_type(c16, jnp.uint32)

    gt = y > u
    n_gt = jnp.sum(gt.astype(jnp.int32), axis=1, keepdims=True)
    r = _K - n_gt
    eq = y == u
    n_eq = jnp.sum(eq.astype(jnp.int32), axis=1, keepdims=True)
    col = lax.broadcasted_iota(jnp.int32, x.shape, 1)

    def col_search(_):
        def cs_body(i, t):
            bit = lax.shift_left(jnp.int32(1), (16 - i))
            cand = t | bit
            cnt = jnp.sum((eq & (col <= cand)).astype(jnp.int32),
                          axis=1, keepdims=True)
            return jnp.where(cnt <= r, cand, t)

        return lax.fori_loop(0, 17, cs_body, jnp.zeros((rows, 1), jnp.int32))

    t = lax.cond(jnp.any(n_eq != r), col_search,
                 lambda _: jnp.full((rows, 1), jnp.int32(0x7FFFFFFF)), None)

    sel = gt | (eq & (col <= t))
    m = jnp.max(x, axis=1, keepdims=True)
    e = jnp.where(sel, jnp.exp(x - m), jnp.float32(0.0))
    z = jnp.sum(e, axis=1, keepdims=True)
    probs_ref[...] = e / z

    row0 = pl.program_id(0) * rows
    rowid = row0 + lax.broadcasted_iota(jnp.int32, (rows, 1), 0)
    p = (rowid * vocab + col).astype(jnp.uint32)
    bits = _threefry_bits(p)
    fb = (bits >> jnp.uint32(9)) | jnp.uint32(0x3F800000)
    f = lax.bitcast_convert_type(fb, jnp.float32) - jnp.float32(1.0)
    tiny = jnp.float32(1.1754943508222875e-38)
    u01 = jnp.maximum(tiny, f * (jnp.float32(1.0) - tiny) + tiny)
    g = -jnp.log(-jnp.log(u01))

    score = jnp.where(sel, x + g, -jnp.inf)
    smax = jnp.max(score, axis=1, keepdims=True)
    tok = jnp.min(jnp.where(score == smax, col, jnp.int32(0x7FFFFFFF)),
                  axis=1, keepdims=True)
    tok_ref[...] = tok


def kernel(logits, k):
    del k
    b, vocab = logits.shape
    rows = _ROWS_PER_STEP
    grid = b // rows
    probs, tok = pl.pallas_call(
        functools.partial(_step, vocab=vocab, rows=rows),
        grid=(grid,),
        in_specs=[pl.BlockSpec((rows, vocab), lambda i: (i, 0))],
        out_specs=[
            pl.BlockSpec((rows, vocab), lambda i: (i, 0)),
            pl.BlockSpec((rows, 1), lambda i: (i, 0)),
        ],
        out_shape=[
            jax.ShapeDtypeStruct((b, vocab), jnp.float32),
            jax.ShapeDtypeStruct((b, 1), jnp.int32),
        ],
    )(logits)
    return tok.reshape(b), probs


# speedup vs baseline: 2.5642x; 1.0222x over previous
import functools

import jax
import jax.numpy as jnp
from jax import lax
from jax.experimental import pallas as pl
from jax.experimental.pallas import tpu as pltpu

_K = 64
_ROWS_PER_STEP = 16


def _threefry_bits(p):
    ks = (jnp.uint32(0), jnp.uint32(1), jnp.uint32(0x1BD11BDA ^ 1))
    rotations = ((13, 15, 26, 6), (17, 29, 16, 24))
    x0 = jnp.zeros_like(p)
    x1 = p + ks[1]
    for i in range(5):
        for r in rotations[i % 2]:
            x0 = x0 + x1
            x1 = (x1 << jnp.uint32(r)) | (x1 >> jnp.uint32(32 - r))
            x1 = x1 ^ x0
        x0 = x0 + ks[(i + 1) % 3]
        x1 = x1 + ks[(i + 2) % 3] + jnp.uint32(i + 1)
    return x0 ^ x1


def _count_ge(v16, cand16, rows, pcols):
    msk = (v16 >= cand16).astype(jnp.int16)
    accs = [jnp.zeros((rows, 256), jnp.int16) for _ in range(4)]
    for j in range(pcols // 256):
        accs[j % 4] = accs[j % 4] + msk[:, j * 256:(j + 1) * 256]
    part = (accs[0] + accs[1]) + (accs[2] + accs[3])
    return jnp.sum(part.astype(jnp.int32), axis=1, keepdims=True)


def _step(x_ref, probs_ref, tok_ref, *, vocab, rows):
    x = x_ref[...]
    xu = lax.bitcast_convert_type(x, jnp.uint32)
    y = jnp.where((xu >> jnp.uint32(31)) == 0, xu | jnp.uint32(0x80000000), ~xu)

    pcols = -(-vocab // 256) * 256
    pad = pcols - vocab
    bias = jnp.int32(0x8000)

    def pack16(v32):
        v = jnp.concatenate(
            [v32, jnp.zeros((rows, pad), jnp.int32)], axis=1)
        return (v ^ bias).astype(jnp.int16)

    hi = lax.bitcast_convert_type(y >> jnp.uint32(16), jnp.int32)
    y16 = pack16(hi)

    def p1_body(i, p):
        cand = p | lax.shift_left(jnp.int32(1), 15 - i)
        cnt = _count_ge(y16, (cand ^ bias).astype(jnp.int16),
                        rows, pcols)
        return jnp.where(cnt >= _K, cand, p)

    p16 = lax.fori_loop(0, 16, p1_body, jnp.zeros((rows, 1), jnp.int32))

    p16b = (p16 ^ bias).astype(jnp.int16)
    n_above = jnp.sum((y16 > p16b).astype(jnp.int32), axis=1, keepdims=True)
    eq16 = y16 == p16b
    lo = lax.bitcast_convert_type(y & jnp.uint32(0xFFFF), jnp.int32)
    key16 = pack16(jnp.where(eq16[:, :vocab], lo, jnp.int32(0)))

    def p2_body(i, c):
        cand = c | lax.shift_left(jnp.int32(1), 15 - i)
        cnt = n_above + _count_ge(
            key16, (cand ^ bias).astype(jnp.int16), rows, pcols)
        return jnp.where(cnt >= _K, cand, c)

    c16 = lax.fori_loop(0, 16, p2_body, jnp.zeros((rows, 1), jnp.int32))

    u = (lax.bitcast_convert_type(p16, jnp.uint32) << jnp.uint32(16)) | \
        lax.bitcast_convert_type(c16, jnp.uint32)

    gt = y > u
    n_gt = jnp.sum(gt.astype(jnp.int32), axis=1, keepdims=True)
    r = _K - n_gt
    eq = y == u
    n_eq = jnp.sum(eq.astype(jnp.int32), axis=1, keepdims=True)
    col = lax.broadcasted_iota(jnp.int32, x.shape, 1)

    def col_search(_):
        def cs_body(i, t):
            bit = lax.shift_left(jnp.int32(1), (16 - i))
            cand = t | bit
            cnt = jnp.sum((eq & (col <= cand)).astype(jnp.int32),
                          axis=1, keepdims=True)
            return jnp.where(cnt <= r, cand, t)

        return lax.fori_loop(0, 17, cs_body, jnp.zeros((rows, 1), jnp.int32))

    t = lax.cond(jnp.any(n_eq != r), col_search,
                 lambda _: jnp.full((rows, 1), jnp.int32(0x7FFFFFFF)), None)

    sel = gt | (eq & (col <= t))
    m = jnp.max(x, axis=1, keepdims=True)
    e = jnp.where(sel, jnp.exp(x - m), jnp.float32(0.0))
    z = jnp.sum(e, axis=1, keepdims=True)
    probs_ref[...] = e / z

    row0 = pl.program_id(0) * rows
    rowid = row0 + lax.broadcasted_iota(jnp.int32, (rows, 1), 0)
    p = (rowid * vocab + col).astype(jnp.uint32)
    bits = _threefry_bits(p)
    fb = (bits >> jnp.uint32(9)) | jnp.uint32(0x3F800000)
    f = lax.bitcast_convert_type(fb, jnp.float32) - jnp.float32(1.0)
    tiny = jnp.float32(1.1754943508222875e-38)
    u01 = jnp.maximum(tiny, f * (jnp.float32(1.0) - tiny) + tiny)
    g = -jnp.log(-jnp.log(u01))

    score = jnp.where(sel, x + g, -jnp.inf)
    smax = jnp.max(score, axis=1, keepdims=True)
    tok = jnp.min(jnp.where(score == smax, col, jnp.int32(0x7FFFFFFF)),
                  axis=1, keepdims=True)
    tok_ref[...] = tok


def kernel(logits, k):
    del k
    b, vocab = logits.shape
    rows = _ROWS_PER_STEP
    grid = b // rows
    probs, tok = pl.pallas_call(
        functools.partial(_step, vocab=vocab, rows=rows),
        grid=(grid,),
        in_specs=[pl.BlockSpec((rows, vocab), lambda i: (i, 0))],
        out_specs=[
            pl.BlockSpec((rows, vocab), lambda i: (i, 0)),
            pl.BlockSpec((rows, 1), lambda i: (i, 0)),
        ],
        out_shape=[
            jax.ShapeDtypeStruct((b, vocab), jnp.float32),
            jax.ShapeDtypeStruct((b, 1), jnp.int32),
        ],
    )(logits)
    return tok.reshape(b), probs
